# baseline (speedup 1.0000x reference)
import functools

import jax
import jax.numpy as jnp
from jax import lax
from jax.experimental import pallas as pl
from jax.experimental.pallas import tpu as pltpu
from jax.experimental.pallas import tpu_sc as plsc

NBUF = 4


@functools.lru_cache(maxsize=None)
def _embed_lookup(B: int, H: int, V: int, D: int):
    info = plsc.get_sparse_core_info()
    nw = info.num_cores * info.num_subcores
    rows_per_w = B // nw
    n_outer = rows_per_w // NBUF
    assert rows_per_w * nw == B and n_outer * NBUF == rows_per_w

    mesh = plsc.VectorSubcoreMesh(core_axis_name="c", subcore_axis_name="s")

    @functools.partial(
        pl.kernel,
        mesh=mesh,
        out_type=jax.ShapeDtypeStruct((B, H, D), jnp.float32),
        scratch_types=[
            pltpu.VMEM((rows_per_w, H), jnp.int32),
            pltpu.VMEM((NBUF, H, D), jnp.float32),
        ]
        + [pltpu.SemaphoreType.DMA] * (2 * NBUF),
        compiler_params=pltpu.CompilerParams(use_tc_tiling_on_sc=False),
    )
    def k(idx_hbm, table_hbm, out_hbm, idx_v, rows_v, *sems):
        gsems, wsems = sems[:NBUF], sems[NBUF:]
        wid = lax.axis_index("s") * info.num_cores + lax.axis_index("c")
        r0 = wid * rows_per_w

        pltpu.sync_copy(idx_hbm.at[pl.ds(r0, rows_per_w)], idx_v)

        def fire_gather(r, b):
            pltpu.async_copy(table_hbm.at[idx_v.at[r]], rows_v.at[b], gsems[b])

        def wait_gather(b):
            pltpu.make_async_copy(
                table_hbm.at[idx_v.at[0]], rows_v.at[b], gsems[b]
            ).wait()

        def fire_write(r, b):
            pltpu.async_copy(rows_v.at[b], out_hbm.at[r0 + r], wsems[b])

        def wait_write(b):
            pltpu.make_async_copy(rows_v.at[b], out_hbm.at[r0], wsems[b]).wait()

        for b in range(NBUF):
            fire_gather(b, b)
        for b in range(NBUF):
            wait_gather(b)
            fire_write(b, b)

        def body(i, carry):
            for b in range(NBUF):
                wait_write(b)
                fire_gather(NBUF * i + b, b)
            for b in range(NBUF):
                wait_gather(b)
                fire_write(NBUF * i + b, b)
            return carry

        lax.fori_loop(1, n_outer, body, 0, unroll=False)

        for b in range(NBUF):
            wait_write(b)

    return k


def kernel(inputs, table):
    b, h = inputs.shape
    v, d = table.shape
    return _embed_lookup(b, h, v, d)(inputs.astype(jnp.int32), table)
